# baseline (speedup 1.0000x reference)
import jax
import jax.numpy as jnp
from jax.experimental import pallas as pl


_BK = 256


def _router_body(x_ref, w_ref, out_ref):
    i = pl.program_id(0)
    partial = jax.lax.dot_general(
        x_ref[...],
        w_ref[...],
        dimension_numbers=(((1,), (1,)), ((), ())),
        preferred_element_type=jnp.float32,
    )

    @pl.when(i == 0)
    def _():
        out_ref[...] = partial

    @pl.when(i > 0)
    def _():
        out_ref[...] += partial


def kernel(x, W):
    m, k = x.shape
    e = W.shape[0]
    return pl.pallas_call(
        _router_body,
        grid=(k // _BK,),
        in_specs=[
            pl.BlockSpec((m, _BK), lambda i: (0, i)),
            pl.BlockSpec((e, _BK), lambda i: (0, i)),
        ],
        out_specs=pl.BlockSpec((m, e), lambda i: (0, 0)),
        out_shape=jax.ShapeDtypeStruct((m, e), jnp.float32),
    )(x, W)
